# baseline (speedup 1.0000x reference)
import functools

import jax
import jax.numpy as jnp
from jax import lax
from jax.experimental import pallas as pl
from jax.experimental.pallas import tpu as pltpu
from jax.experimental.pallas import tpu_sc as plsc

B, L, D = 2, 2048, 2048
E = 64
H = 256
K = 16
G = 64
T = B * L
TQ = T // G
NBLK = TQ * K
BPT = 4
NSLOT = NBLK + E * (BPT - 1) + 8
NT = NSLOT // BPT
MT = BPT * G
CAP = NSLOT * G

_NC, _NS = 2, 16
_NW = _NC * _NS

_GROWS = CAP // _NW
_GCH = 24
_GN = _GROWS // _GCH

_CPARTS = 8
_PC = D // _CPARTS
_PPS = _CPARTS // _NC
_CROWS = CAP // _NS
_CCH = 96
_CN = _CROWS // _CCH
_ORPT = T // _NS


def _routing(x, W_router, b_router):
    xt = x.reshape(T, D)
    logits = xt @ W_router.T + b_router
    scores = jax.nn.softmax(logits.astype(jnp.float32), axis=-1)
    order = jnp.argsort(-scores, axis=0)
    sorted_vals = jnp.take_along_axis(scores, order, axis=0)
    group_score = sorted_vals.reshape(TQ, G, E).sum(axis=1)
    _, top_idx = jax.lax.top_k(group_score.reshape(-1), NBLK)
    mask_group = (
        jnp.zeros((TQ * E,), dtype=bool).at[top_idx].set(True).reshape(TQ, E)
    )
    return order, sorted_vals, mask_group


def _dispatch_meta(order, sorted_vals, mask_group):
    n_e = mask_group.sum(axis=0).astype(jnp.int32)
    blk = jnp.nonzero(mask_group.T.reshape(-1), size=NBLK, fill_value=0)[0]
    blk = blk.astype(jnp.int32)
    e_b = blk // TQ
    q_b = blk % TQ
    npad_e = ((n_e + BPT - 1) // BPT) * BPT
    ends = jnp.cumsum(npad_e)
    offs_e = ends - npad_e
    cstart = jnp.cumsum(n_e) - n_e
    dst = offs_e[e_b] + (jnp.arange(NBLK, dtype=jnp.int32) - cstart[e_b])
    slot_q = jnp.zeros((NSLOT,), jnp.int32).at[dst].set(q_b)
    slot_valid = jnp.zeros((NSLOT,), bool).at[dst].set(True)
    tile_expert = jnp.searchsorted(
        ends, jnp.arange(0, NSLOT, BPT, dtype=jnp.int32), side="right"
    ).astype(jnp.int32)
    tile_expert = jnp.minimum(tile_expert, E - 1)
    slot_e = jnp.repeat(tile_expert, BPT)
    lane = jnp.arange(G, dtype=jnp.int32)[None, :]
    flat = slot_e[:, None] * T + slot_q[:, None] * G + lane
    ordT = order.T.reshape(-1).astype(jnp.int32)
    svT = sorted_vals.T.reshape(-1)
    varied = jnp.arange(CAP, dtype=jnp.int32).reshape(NSLOT, G) % T
    tok = jnp.where(slot_valid[:, None], ordT[flat], varied).reshape(CAP)
    gate = jnp.where(slot_valid[:, None], svT[flat], 0.0).reshape(CAP)
    return tile_expert, tok, gate


_DP = D // 2


def _sc_gather(x2, tok):
    mesh = plsc.VectorSubcoreMesh(core_axis_name="c", subcore_axis_name="s")

    @functools.partial(
        pl.kernel,
        out_type=jax.ShapeDtypeStruct((CAP, _DP), jnp.int32),
        mesh=mesh,
        scratch_types=[
            pltpu.VMEM((_GROWS,), jnp.int32),
            pltpu.VMEM((_GCH, _DP), jnp.int32),
            pltpu.VMEM((_GCH, _DP), jnp.int32),
            pltpu.SemaphoreType.DMA,
            pltpu.SemaphoreType.DMA,
            pltpu.SemaphoreType.DMA,
            pltpu.SemaphoreType.DMA,
        ],
    )
    def k(x_hbm, tok_hbm, xs_hbm, idx_v, buf0, buf1, g0, g1, w0, w1):
        wid = lax.axis_index("s") * _NC + lax.axis_index("c")
        base0 = wid * _GROWS
        pltpu.sync_copy(tok_hbm.at[pl.ds(base0, _GROWS)], idx_v)

        def gather(i, buf, sem):
            pltpu.async_copy(x_hbm.at[idx_v.at[pl.ds(i * _GCH, _GCH)]], buf, sem)

        def gwait(buf, sem):
            pltpu.make_async_copy(x_hbm.at[idx_v.at[pl.ds(0, _GCH)]], buf, sem).wait()

        def wback(i, buf, sem):
            pltpu.async_copy(buf, xs_hbm.at[pl.ds(base0 + i * _GCH, _GCH)], sem)

        def wwait(buf, sem):
            pltpu.make_async_copy(buf, xs_hbm.at[pl.ds(base0, _GCH)], sem).wait()

        gather(0, buf0, g0)
        gather(1, buf1, g1)
        gwait(buf0, g0)
        wback(0, buf0, w0)
        gwait(buf1, g1)
        wback(1, buf1, w1)

        def pair(j, _):
            i0 = 2 * j
            pltpu.make_async_copy(buf0, xs_hbm.at[pl.ds(base0, _GCH)], w0).wait()
            gather(i0, buf0, g0)
            gwait(buf0, g0)
            wback(i0, buf0, w0)
            pltpu.make_async_copy(buf1, xs_hbm.at[pl.ds(base0, _GCH)], w1).wait()
            gather(i0 + 1, buf1, g1)
            gwait(buf1, g1)
            wback(i0 + 1, buf1, w1)
            return ()

        lax.fori_loop(1, _GN // 2, pair, ())
        wwait(buf0, w0)
        wwait(buf1, w1)

    return k(x2, tok)


def _sc_combine(os, tok, zrows):
    mesh = plsc.VectorSubcoreMesh(core_axis_name="c", subcore_axis_name="s")

    @functools.partial(
        pl.kernel,
        out_type=jax.ShapeDtypeStruct((T, D), jnp.float32),
        mesh=mesh,
        scratch_types=[
            pltpu.VMEM((_CCH,), jnp.int32),
            pltpu.VMEM((_CCH, _PC), jnp.float32),
            pltpu.VMEM_SHARED((T, _PC), jnp.float32),
            pltpu.SemaphoreType.DMA,
        ],
    )
    def k(os_hbm, tok_hbm, z_hbm, out_hbm, idx_v, stage_v, accum, sem):
        cid = lax.axis_index("c")
        sid = lax.axis_index("s")
        rbase = sid * _CROWS
        obase = sid * _ORPT

        for j in range(_PPS):
            part = cid * _PPS + j
            cbase = part * _PC
            pltpu.sync_copy(z_hbm, accum.at[pl.ds(obase, _ORPT)])
            plsc.subcore_barrier()

            def batch(i, _):
                base = rbase + i * _CCH
                pltpu.sync_copy(tok_hbm.at[pl.ds(base, _CCH)], idx_v)
                pltpu.sync_copy(
                    os_hbm.at[pl.ds(base, _CCH), pl.ds(cbase, _PC)], stage_v
                )
                pltpu.async_copy(stage_v, accum.at[idx_v], sem, add=True).wait()
                return ()

            lax.fori_loop(0, _CN, batch, ())
            plsc.subcore_barrier()
            pltpu.sync_copy(
                accum.at[pl.ds(obase, _ORPT)],
                out_hbm.at[pl.ds(obase, _ORPT), pl.ds(cbase, _PC)],
            )
            plsc.subcore_barrier()

    return k(os, tok, zrows)


def _tc_body(te_ref, xs_ref, gate_ref, wgu_ref, bgu_ref, wd_ref, bd_ref, os_ref):
    xt = xs_ref.bitcast(jnp.bfloat16)[...].reshape(MT, D)
    h = jnp.dot(xt, wgu_ref[0], preferred_element_type=jnp.float32)
    h = h + bgu_ref[0]
    g = h[:, :H]
    u = h[:, H:]
    a = (g * jax.nn.sigmoid(g) * u).astype(jnp.bfloat16)
    o = jnp.dot(a, wd_ref[0], preferred_element_type=jnp.float32)
    o = o + bd_ref[0]
    gate = gate_ref[0, 0]
    os_ref[...] = o * gate[:, None]


def _tc_expert(xs_bf, gate3, tile_expert, Wgu_t, bgu, Wd_t, bd2):
    grid_spec = pltpu.PrefetchScalarGridSpec(
        num_scalar_prefetch=1,
        grid=(NT,),
        in_specs=[
            pl.BlockSpec((MT, _DP), lambda i, te: (i, 0)),
            pl.BlockSpec((1, 1, MT), lambda i, te: (i, 0, 0)),
            pl.BlockSpec((1, D, 2 * H), lambda i, te: (te[i], 0, 0)),
            pl.BlockSpec((1, 1, 2 * H), lambda i, te: (te[i], 0, 0)),
            pl.BlockSpec((1, H, D), lambda i, te: (te[i], 0, 0)),
            pl.BlockSpec((1, 1, D), lambda i, te: (te[i], 0, 0)),
        ],
        out_specs=pl.BlockSpec((MT, D), lambda i, te: (i, 0)),
    )
    return pl.pallas_call(
        _tc_body,
        grid_spec=grid_spec,
        out_shape=jax.ShapeDtypeStruct((CAP, D), jnp.float32),
        compiler_params=pltpu.CompilerParams(
            dimension_semantics=("arbitrary",),
        ),
    )(tile_expert, xs_bf, gate3, Wgu_t, bgu, Wd_t, bd2)


def kernel(x, W_router, b_router, Wg, bg, Wu, bu, Wd, bd):
    order, sorted_vals, mask_group = _routing(x, W_router, b_router)
    tile_expert, tok, gate = _dispatch_meta(order, sorted_vals, mask_group)
    denom = jnp.zeros((T,), jnp.float32).at[tok].add(gate) + 1e-6

    x_pack = jax.lax.bitcast_convert_type(
        x.reshape(T, D).astype(jnp.bfloat16).reshape(T, _DP, 2), jnp.int32
    )
    xs = _sc_gather(x_pack, tok)

    perm = jnp.concatenate(
        [jnp.arange(0, D, 2, dtype=jnp.int32), jnp.arange(1, D, 2, dtype=jnp.int32)]
    )
    Wgu_t = (
        jnp.concatenate([Wg, Wu], axis=1).transpose(0, 2, 1).astype(jnp.bfloat16)
    )[:, perm, :]

    bgu = jnp.concatenate([bg, bu], axis=1)[:, None, :]
    Wd_t = Wd.transpose(0, 2, 1).astype(jnp.bfloat16)
    bd2 = bd[:, None, :]
    gate3 = gate.reshape(NT, 1, MT)

    os = _tc_expert(xs, gate3, tile_expert, Wgu_t, bgu, Wd_t, bd2)

    acc = jnp.zeros((T, D), jnp.float32).at[tok].add(os)
    out = acc / denom[:, None]
    return out.reshape(B, L, D).astype(x.dtype)


# speedup vs baseline: 1.0982x; 1.0982x over previous
import functools

import jax
import jax.numpy as jnp
from jax import lax
from jax.experimental import pallas as pl
from jax.experimental.pallas import tpu as pltpu
from jax.experimental.pallas import tpu_sc as plsc

B, L, D = 2, 2048, 2048
E = 64
H = 256
K = 16
G = 64
T = B * L
TQ = T // G
NBLK = TQ * K
BPT = 4
NSLOT = NBLK + E * (BPT - 1) + 8
NT = NSLOT // BPT
MT = BPT * G
CAP = NSLOT * G

_NC, _NS = 2, 16
_NW = _NC * _NS

_GROWS = CAP // _NW
_GCH = 24
_GN = _GROWS // _GCH

def _routing(x, W_router, b_router):
    xt = x.reshape(T, D)
    logits = xt @ W_router.T + b_router
    scores = jax.nn.softmax(logits.astype(jnp.float32), axis=-1)
    order = jnp.argsort(-scores, axis=0)
    sorted_vals = jnp.take_along_axis(scores, order, axis=0)
    group_score = sorted_vals.reshape(TQ, G, E).sum(axis=1)
    _, top_idx = jax.lax.top_k(group_score.reshape(-1), NBLK)
    mask_group = (
        jnp.zeros((TQ * E,), dtype=bool).at[top_idx].set(True).reshape(TQ, E)
    )
    return order, sorted_vals, mask_group


def _dispatch_meta(order, sorted_vals, mask_group):
    n_e = mask_group.sum(axis=0).astype(jnp.int32)
    blk = jnp.nonzero(mask_group.T.reshape(-1), size=NBLK, fill_value=0)[0]
    blk = blk.astype(jnp.int32)
    e_b = blk // TQ
    q_b = blk % TQ
    npad_e = ((n_e + BPT - 1) // BPT) * BPT
    ends = jnp.cumsum(npad_e)
    offs_e = ends - npad_e
    cstart = jnp.cumsum(n_e) - n_e
    dst = offs_e[e_b] + (jnp.arange(NBLK, dtype=jnp.int32) - cstart[e_b])
    slot_q = jnp.zeros((NSLOT,), jnp.int32).at[dst].set(q_b)
    slot_valid = jnp.zeros((NSLOT,), bool).at[dst].set(True)
    tile_expert = jnp.searchsorted(
        ends, jnp.arange(0, NSLOT, BPT, dtype=jnp.int32), side="right"
    ).astype(jnp.int32)
    tile_expert = jnp.minimum(tile_expert, E - 1)
    slot_e = jnp.repeat(tile_expert, BPT)
    lane = jnp.arange(G, dtype=jnp.int32)[None, :]
    flat = slot_e[:, None] * T + slot_q[:, None] * G + lane
    ordT = order.T.reshape(-1).astype(jnp.int32)
    svT = sorted_vals.T.reshape(-1)
    varied = jnp.arange(CAP, dtype=jnp.int32).reshape(NSLOT, G) % T
    tok = jnp.where(slot_valid[:, None], ordT[flat], varied).reshape(CAP)
    gate = jnp.where(slot_valid[:, None], svT[flat], 0.0).reshape(CAP)
    return tile_expert, tok, gate


_DP = D // 2


def _sc_gather(x2, tok):
    mesh = plsc.VectorSubcoreMesh(core_axis_name="c", subcore_axis_name="s")

    @functools.partial(
        pl.kernel,
        out_type=jax.ShapeDtypeStruct((CAP, _DP), jnp.int32),
        mesh=mesh,
        scratch_types=[
            pltpu.VMEM((_GROWS,), jnp.int32),
            pltpu.VMEM((_GCH, _DP), jnp.int32),
            pltpu.VMEM((_GCH, _DP), jnp.int32),
            pltpu.SemaphoreType.DMA,
            pltpu.SemaphoreType.DMA,
            pltpu.SemaphoreType.DMA,
            pltpu.SemaphoreType.DMA,
        ],
    )
    def k(x_hbm, tok_hbm, xs_hbm, idx_v, buf0, buf1, g0, g1, w0, w1):
        wid = lax.axis_index("s") * _NC + lax.axis_index("c")
        base0 = wid * _GROWS
        pltpu.sync_copy(tok_hbm.at[pl.ds(base0, _GROWS)], idx_v)

        def gather(i, buf, sem):
            pltpu.async_copy(x_hbm.at[idx_v.at[pl.ds(i * _GCH, _GCH)]], buf, sem)

        def gwait(buf, sem):
            pltpu.make_async_copy(x_hbm.at[idx_v.at[pl.ds(0, _GCH)]], buf, sem).wait()

        def wback(i, buf, sem):
            pltpu.async_copy(buf, xs_hbm.at[pl.ds(base0 + i * _GCH, _GCH)], sem)

        def wwait(buf, sem):
            pltpu.make_async_copy(buf, xs_hbm.at[pl.ds(base0, _GCH)], sem).wait()

        gather(0, buf0, g0)
        gather(1, buf1, g1)
        gwait(buf0, g0)
        wback(0, buf0, w0)
        gwait(buf1, g1)
        wback(1, buf1, w1)

        def pair(j, _):
            i0 = 2 * j
            pltpu.make_async_copy(buf0, xs_hbm.at[pl.ds(base0, _GCH)], w0).wait()
            gather(i0, buf0, g0)
            gwait(buf0, g0)
            wback(i0, buf0, w0)
            pltpu.make_async_copy(buf1, xs_hbm.at[pl.ds(base0, _GCH)], w1).wait()
            gather(i0 + 1, buf1, g1)
            gwait(buf1, g1)
            wback(i0 + 1, buf1, w1)
            return ()

        lax.fori_loop(1, _GN // 2, pair, ())
        wwait(buf0, w0)
        wwait(buf1, w1)

    return k(x2, tok)


def _tc_body(te_ref, xs_ref, gate_ref, wgu_ref, bgu_ref, wd_ref, bd_ref, os_ref):
    xt = xs_ref.bitcast(jnp.bfloat16)[...].reshape(MT, D)
    h = jnp.dot(xt, wgu_ref[0], preferred_element_type=jnp.float32)
    h = h + bgu_ref[0]
    g = h[:, :H]
    u = h[:, H:]
    a = (g * jax.nn.sigmoid(g) * u).astype(jnp.bfloat16)
    o = jnp.dot(a, wd_ref[0], preferred_element_type=jnp.float32)
    o = o + bd_ref[0]
    gate = gate_ref[0, 0]
    os_ref[...] = o * gate[:, None]


def _tc_expert(xs_bf, gate3, tile_expert, Wgu_t, bgu, Wd_t, bd2):
    grid_spec = pltpu.PrefetchScalarGridSpec(
        num_scalar_prefetch=1,
        grid=(NT,),
        in_specs=[
            pl.BlockSpec((MT, _DP), lambda i, te: (i, 0)),
            pl.BlockSpec((1, 1, MT), lambda i, te: (i, 0, 0)),
            pl.BlockSpec((1, D, 2 * H), lambda i, te: (te[i], 0, 0)),
            pl.BlockSpec((1, 1, 2 * H), lambda i, te: (te[i], 0, 0)),
            pl.BlockSpec((1, H, D), lambda i, te: (te[i], 0, 0)),
            pl.BlockSpec((1, 1, D), lambda i, te: (te[i], 0, 0)),
        ],
        out_specs=pl.BlockSpec((MT, D), lambda i, te: (i, 0)),
    )
    return pl.pallas_call(
        _tc_body,
        grid_spec=grid_spec,
        out_shape=jax.ShapeDtypeStruct((CAP, D), jnp.float32),
        compiler_params=pltpu.CompilerParams(
            dimension_semantics=("arbitrary",),
        ),
    )(tile_expert, xs_bf, gate3, Wgu_t, bgu, Wd_t, bd2)


def kernel(x, W_router, b_router, Wg, bg, Wu, bu, Wd, bd):
    order, sorted_vals, mask_group = _routing(x, W_router, b_router)
    tile_expert, tok, gate = _dispatch_meta(order, sorted_vals, mask_group)
    denom = jnp.zeros((T,), jnp.float32).at[tok].add(gate) + 1e-6

    x_bf = x.reshape(T, D).astype(jnp.bfloat16)
    x_pack = jax.lax.bitcast_convert_type(
        jnp.stack([x_bf[:, :_DP], x_bf[:, _DP:]], axis=-1), jnp.int32
    )
    xs = _sc_gather(x_pack, tok)

    Wgu_t = (
        jnp.concatenate([Wg, Wu], axis=1).transpose(0, 2, 1).astype(jnp.bfloat16)
    )

    bgu = jnp.concatenate([bg, bu], axis=1)[:, None, :]
    Wd_t = Wd.transpose(0, 2, 1).astype(jnp.bfloat16)
    bd2 = bd[:, None, :]
    gate3 = gate.reshape(NT, 1, MT)

    os = _tc_expert(xs, gate3, tile_expert, Wgu_t, bgu, Wd_t, bd2)

    acc = jnp.zeros((T, D), jnp.float32).at[tok].add(os)
    out = acc / denom[:, None]
    return out.reshape(B, L, D).astype(x.dtype)
